# baseline (speedup 1.0000x reference)
import functools

import jax
import jax.numpy as jnp
from jax import lax
from jax.experimental import pallas as pl
from jax.experimental.pallas import tpu as pltpu
from jax.experimental.pallas import tpu_sc as plsc

B, S, H, F, E, K, NS = 2, 2048, 1024, 2048, 8, 2, 2
T = B * S
A = T * K
BLK = 256
G = (A + E * (BLK - 1)) // BLK + 1
NPAD = G * BLK
FC = 1024


def _router_body(x_ref, wr_ref, br_ref, scores_ref, tv_ref, ti_ref):
    x = x_ref[...]
    scores = jnp.dot(x, wr_ref[...], preferred_element_type=jnp.float32)
    scores = scores + br_ref[...]
    scores_ref[...] = scores
    m = jnp.max(scores, axis=-1, keepdims=True)
    ex = jnp.exp(scores - m)
    probs = ex / jnp.sum(ex, axis=-1, keepdims=True)
    cols = lax.broadcasted_iota(jnp.int32, probs.shape, 1)
    v1 = jnp.max(probs, axis=-1, keepdims=True)
    i1 = jnp.min(jnp.where(probs == v1, cols, jnp.int32(E)), axis=-1,
                 keepdims=True)
    probs2 = jnp.where(cols == i1, jnp.float32(-jnp.inf), probs)
    v2 = jnp.max(probs2, axis=-1, keepdims=True)
    i2 = jnp.min(jnp.where(probs2 == v2, cols, jnp.int32(E)), axis=-1,
                 keepdims=True)
    tv_ref[...] = jnp.concatenate([v1, v2], axis=1)
    ti_ref[...] = jnp.concatenate([i1, i2], axis=1)


def _bucket_body(ids_ref, slot_ref, eid_ref):
    ids = ids_ref[...]
    r_ = lax.broadcasted_iota(jnp.int32, (128, 128), 0)
    c_ = lax.broadcasted_iota(jnp.int32, (128, 128), 1)
    ustrict = jnp.where(r_ < c_, 1.0, 0.0)
    r64 = lax.broadcasted_iota(jnp.int32, (64, 64), 0)
    c64 = lax.broadcasted_iota(jnp.int32, (64, 64), 1)
    lstrict = jnp.where(c64 < r64, 1.0, 0.0)

    counts = []
    masks = []
    for e in range(E):
        m = (ids == e).astype(jnp.float32)
        masks.append(m)
        counts.append(jnp.sum(m))
    offs = []
    cums = []
    acc = jnp.float32(0.0)
    for e in range(E):
        offs.append(acc)
        pc = jnp.floor((counts[e] + (BLK - 1)) / BLK) * BLK
        acc = acc + pc
        cums.append(acc)

    slot = jnp.zeros((64, 128), jnp.float32)
    for e in range(E):
        m = masks[e]
        within = jnp.dot(m, ustrict, preferred_element_type=jnp.float32)
        s = jnp.sum(m, axis=1, keepdims=True)
        p = jnp.dot(lstrict, s, preferred_element_type=jnp.float32)
        rank = within + p
        slot = slot + m * (offs[e] + rank)
    slot_ref[...] = slot.astype(jnp.int32)

    gbase = lax.broadcasted_iota(jnp.int32, (1, 128), 1).astype(jnp.float32)
    gbase = gbase * BLK
    eid = jnp.zeros((1, 128), jnp.float32)
    for e in range(E):
        eid = eid + jnp.where(gbase >= cums[e], 1.0, 0.0)
    eid_ref[...] = jnp.minimum(eid, E - 1).astype(jnp.int32)


NW = 32
_GCH = 32


def _sc_wid():
    return lax.axis_index("s") * 2 + lax.axis_index("c")


@functools.lru_cache(maxsize=1)
def _sc_kernels():
    mesh = plsc.VectorSubcoreMesh(core_axis_name="c", subcore_axis_name="s")

    @functools.partial(
        pl.kernel, mesh=mesh,
        out_type=jax.ShapeDtypeStruct((NPAD, H), jnp.float32),
        scratch_types=[
            pltpu.VMEM((2, _GCH), jnp.int32),
            pltpu.VMEM((2, _GCH), jnp.int32),
            pltpu.VMEM((2, _GCH, H), jnp.float32),
            pltpu.SemaphoreType.DMA,
            pltpu.SemaphoreType.DMA,
            pltpu.SemaphoreType.DMA,
        ],
    )
    def _sc_scatter_x(x_hbm, slot_hbm, out_hbm, idx0_v, idx1_v, rows_v,
                      lsem, sem0, sem1):
        wid = _sc_wid()
        per = T // NW
        base = wid * per
        nch = per // _GCH
        loads = [None] * nch
        scat = [None] * nch
        for c in range(nch):
            cb = base + c * _GCH
            if c >= 2:
                scat[c - 2][0].wait()
                scat[c - 2][1].wait()
            loads[c] = pltpu.async_copy(x_hbm.at[pl.ds(cb, _GCH)],
                                        rows_v.at[c % 2], lsem)
            pltpu.sync_copy(slot_hbm.at[0, pl.ds(cb, _GCH)], idx0_v.at[c % 2])
            pltpu.sync_copy(slot_hbm.at[1, pl.ds(cb, _GCH)], idx1_v.at[c % 2])
            loads[c].wait()
            scat[c] = (
                pltpu.async_copy(rows_v.at[c % 2], out_hbm.at[idx0_v.at[c % 2]],
                                 sem0),
                pltpu.async_copy(rows_v.at[c % 2], out_hbm.at[idx1_v.at[c % 2]],
                                 sem1),
            )
        for c in range(max(nch - 2, 0), nch):
            scat[c][0].wait()
            scat[c][1].wait()

    @functools.partial(
        pl.kernel, mesh=mesh,
        out_type=jax.ShapeDtypeStruct((A, H), jnp.float32),
        scratch_types=[
            pltpu.VMEM((2, _GCH), jnp.int32),
            pltpu.VMEM((2, _GCH, H), jnp.float32),
            pltpu.SemaphoreType.DMA,
            pltpu.SemaphoreType.DMA,
        ],
    )
    def _sc_gather_pairs(y_hbm, slot_hbm, out_hbm, idx_v, rows_v, gsem, ssem):
        wid = _sc_wid()
        per = T // NW
        nch = per // _GCH
        base = wid * per
        stores = [None] * (K * nch)
        for k in range(K):
            for c in range(nch):
                i = k * nch + c
                cb = base + c * _GCH
                if i >= 2:
                    stores[i - 2].wait()
                pltpu.sync_copy(slot_hbm.at[k, pl.ds(cb, _GCH)],
                                idx_v.at[i % 2])
                pltpu.async_copy(y_hbm.at[idx_v.at[i % 2]], rows_v.at[i % 2],
                                 gsem).wait()
                stores[i] = pltpu.async_copy(
                    rows_v.at[i % 2], out_hbm.at[pl.ds(k * T + cb, _GCH)],
                    ssem)
        for i in range(max(K * nch - 2, 0), K * nch):
            stores[i].wait()

    return _sc_scatter_x, _sc_gather_pairs


def _group_mlp_body(eid_ref, x_ref, wg_ref, wu_ref, wd_ref, y_ref):
    x = x_ref[...]
    g = jnp.dot(x, wg_ref[0], preferred_element_type=jnp.float32)
    u = jnp.dot(x, wu_ref[0], preferred_element_type=jnp.float32)
    a = g * jax.nn.sigmoid(g) * u
    y_ref[...] = jnp.dot(a, wd_ref[0], preferred_element_type=jnp.float32)


def _shared_body(x_ref, wg_ref, wu_ref, wd_ref, out_ref, acc_ref):
    f = pl.program_id(0)
    t = pl.program_id(1)
    x = x_ref[...]
    g = jnp.dot(x, wg_ref[0], preferred_element_type=jnp.float32)
    u = jnp.dot(x, wu_ref[0], preferred_element_type=jnp.float32)
    a = g * jax.nn.sigmoid(g) * u
    y = jnp.dot(a, wd_ref[0], preferred_element_type=jnp.float32)
    base = t * 256
    nf = F // FC

    @pl.when(f == 0)
    def _():
        acc_ref[pl.ds(base, 256), :] = y

    @pl.when(f == nf - 1)
    def _():
        out_ref[...] = acc_ref[pl.ds(base, 256), :] + y


def _combine_body(x_ref, sh0_ref, sh1_ref, yk0_ref, yk1_ref, tv_ref, out_ref):
    tv = tv_ref[...]
    out_ref[...] = (x_ref[...] + sh0_ref[...] + sh1_ref[...]
                    + tv[:, 0:1] * yk0_ref[...] + tv[:, 1:2] * yk1_ref[...])


def kernel(hidden_states, Wr, br, lWg, lWu, lWd, sWg, sWu, sWd):
    flat = hidden_states.reshape(T, H)

    RB = 512
    scores, tv, ti = pl.pallas_call(
        _router_body,
        grid=(T // RB,),
        in_specs=[
            pl.BlockSpec((RB, H), lambda t: (t, 0)),
            pl.BlockSpec((H, E), lambda t: (0, 0)),
            pl.BlockSpec((1, E), lambda t: (0, 0)),
        ],
        out_specs=(
            pl.BlockSpec((RB, E), lambda t: (t, 0)),
            pl.BlockSpec((RB, K), lambda t: (t, 0)),
            pl.BlockSpec((RB, K), lambda t: (t, 0)),
        ),
        out_shape=(
            jax.ShapeDtypeStruct((T, E), jnp.float32),
            jax.ShapeDtypeStruct((T, K), jnp.float32),
            jax.ShapeDtypeStruct((T, K), jnp.int32),
        ),
    )(flat, Wr, br.reshape(1, E))

    ids_r = ti.T.reshape(64, 128)
    slot_r, eid = pl.pallas_call(
        _bucket_body,
        out_shape=(
            jax.ShapeDtypeStruct((64, 128), jnp.int32),
            jax.ShapeDtypeStruct((1, 128), jnp.int32),
        ),
    )(ids_r)

    sc_scatter_x, sc_gather_pairs = _sc_kernels()
    slot_kt = slot_r.reshape(K, T)
    x_sorted = sc_scatter_x(flat, slot_kt)

    TB = 256

    def _shared_call(si):
        return pl.pallas_call(
            _shared_body,
            grid=(F // FC, T // TB),
            in_specs=[
                pl.BlockSpec((TB, H), lambda f, t: (t, 0)),
                pl.BlockSpec((1, H, FC), lambda f, t, si=si: (si, 0, f)),
                pl.BlockSpec((1, H, FC), lambda f, t, si=si: (si, 0, f)),
                pl.BlockSpec((1, FC, H), lambda f, t, si=si: (si, f, 0)),
            ],
            out_specs=pl.BlockSpec((TB, H), lambda f, t: (t, 0)),
            out_shape=jax.ShapeDtypeStruct((T, H), jnp.float32),
            scratch_shapes=[pltpu.VMEM((T, H), jnp.float32)],
        )(flat, sWg, sWu, sWd)

    shared0 = _shared_call(0)

    eid_arr = eid.reshape(128)[:G]
    grid_spec = pltpu.PrefetchScalarGridSpec(
        num_scalar_prefetch=1,
        grid=(G,),
        in_specs=[
            pl.BlockSpec((BLK, H), lambda g, eid_ref: (g, 0)),
            pl.BlockSpec((1, H, F), lambda g, eid_ref: (eid_ref[g], 0, 0)),
            pl.BlockSpec((1, H, F), lambda g, eid_ref: (eid_ref[g], 0, 0)),
            pl.BlockSpec((1, F, H), lambda g, eid_ref: (eid_ref[g], 0, 0)),
        ],
        out_specs=pl.BlockSpec((BLK, H), lambda g, eid_ref: (g, 0)),
    )
    y_sorted = pl.pallas_call(
        _group_mlp_body,
        grid_spec=grid_spec,
        out_shape=jax.ShapeDtypeStruct((NPAD, H), jnp.float32),
    )(eid_arr, x_sorted, lWg, lWu, lWd)

    y_pair = sc_gather_pairs(y_sorted, slot_kt)
    shared1 = _shared_call(1)

    nt = T // TB
    out = pl.pallas_call(
        _combine_body,
        grid=(nt,),
        in_specs=[
            pl.BlockSpec((TB, H), lambda t: (t, 0)),
            pl.BlockSpec((TB, H), lambda t: (t, 0)),
            pl.BlockSpec((TB, H), lambda t: (t, 0)),
            pl.BlockSpec((TB, H), lambda t: (t, 0)),
            pl.BlockSpec((TB, H), lambda t: (t + nt, 0)),
            pl.BlockSpec((TB, K), lambda t: (t, 0)),
        ],
        out_specs=pl.BlockSpec((TB, H), lambda t: (t, 0)),
        out_shape=jax.ShapeDtypeStruct((T, H), jnp.float32),
    )(flat, shared0, shared1, y_pair, y_pair, tv)

    return out.reshape(B, S, H), scores.reshape(B, S, E)


# speedup vs baseline: 1.0321x; 1.0321x over previous
import functools

import jax
import jax.numpy as jnp
from jax import lax
from jax.experimental import pallas as pl
from jax.experimental.pallas import tpu as pltpu
from jax.experimental.pallas import tpu_sc as plsc

B, S, H, F, E, K, NS = 2, 2048, 1024, 2048, 8, 2, 2
T = B * S
A = T * K
BLK = 256
G = (A + E * (BLK - 1)) // BLK + 1
NPAD = G * BLK
FC = 1024


def _router_body(x_ref, wr_ref, br_ref, scores_ref, tv_ref, ti_ref):
    x = x_ref[...]
    scores = jnp.dot(x, wr_ref[...], preferred_element_type=jnp.float32)
    scores = scores + br_ref[...]
    scores_ref[...] = scores
    m = jnp.max(scores, axis=-1, keepdims=True)
    ex = jnp.exp(scores - m)
    probs = ex / jnp.sum(ex, axis=-1, keepdims=True)
    cols = lax.broadcasted_iota(jnp.int32, probs.shape, 1)
    v1 = jnp.max(probs, axis=-1, keepdims=True)
    i1 = jnp.min(jnp.where(probs == v1, cols, jnp.int32(E)), axis=-1,
                 keepdims=True)
    probs2 = jnp.where(cols == i1, jnp.float32(-jnp.inf), probs)
    v2 = jnp.max(probs2, axis=-1, keepdims=True)
    i2 = jnp.min(jnp.where(probs2 == v2, cols, jnp.int32(E)), axis=-1,
                 keepdims=True)
    tv_ref[...] = jnp.concatenate([v1, v2], axis=1)
    ti_ref[...] = jnp.concatenate([i1, i2], axis=1)


def _bucket_body(ids_ref, slot_ref, eid_ref):
    ids = ids_ref[...]
    r_ = lax.broadcasted_iota(jnp.int32, (128, 128), 0)
    c_ = lax.broadcasted_iota(jnp.int32, (128, 128), 1)
    ustrict = jnp.where(r_ < c_, 1.0, 0.0)
    r64 = lax.broadcasted_iota(jnp.int32, (64, 64), 0)
    c64 = lax.broadcasted_iota(jnp.int32, (64, 64), 1)
    lstrict = jnp.where(c64 < r64, 1.0, 0.0)

    counts = []
    masks = []
    for e in range(E):
        m = (ids == e).astype(jnp.float32)
        masks.append(m)
        counts.append(jnp.sum(m))
    offs = []
    cums = []
    acc = jnp.float32(0.0)
    for e in range(E):
        offs.append(acc)
        pc = jnp.floor((counts[e] + (BLK - 1)) / BLK) * BLK
        acc = acc + pc
        cums.append(acc)

    slot = jnp.zeros((64, 128), jnp.float32)
    for e in range(E):
        m = masks[e]
        within = jnp.dot(m, ustrict, preferred_element_type=jnp.float32)
        s = jnp.sum(m, axis=1, keepdims=True)
        p = jnp.dot(lstrict, s, preferred_element_type=jnp.float32)
        rank = within + p
        slot = slot + m * (offs[e] + rank)
    slot_ref[...] = slot.astype(jnp.int32)

    gbase = lax.broadcasted_iota(jnp.int32, (1, 128), 1).astype(jnp.float32)
    gbase = gbase * BLK
    eid = jnp.zeros((1, 128), jnp.float32)
    for e in range(E):
        eid = eid + jnp.where(gbase >= cums[e], 1.0, 0.0)
    eid_ref[...] = jnp.minimum(eid, E - 1).astype(jnp.int32)


NW = 32
_GCH = 32


def _sc_wid():
    return lax.axis_index("s") * 2 + lax.axis_index("c")


@functools.lru_cache(maxsize=1)
def _sc_kernels():
    mesh = plsc.VectorSubcoreMesh(core_axis_name="c", subcore_axis_name="s")

    @functools.partial(
        pl.kernel, mesh=mesh,
        out_type=jax.ShapeDtypeStruct((NPAD, H), jnp.float32),
        scratch_types=[
            pltpu.VMEM((2, _GCH), jnp.int32),
            pltpu.VMEM((2, _GCH), jnp.int32),
            pltpu.VMEM((2, _GCH, H), jnp.float32),
            pltpu.SemaphoreType.DMA,
            pltpu.SemaphoreType.DMA,
            pltpu.SemaphoreType.DMA,
        ],
    )
    def _sc_scatter_x(x_hbm, slot_hbm, out_hbm, idx0_v, idx1_v, rows_v,
                      lsem, sem0, sem1):
        wid = _sc_wid()
        per = T // NW
        base = wid * per
        nch = per // _GCH
        loads = [None] * nch
        scat = [None] * nch
        for c in range(nch):
            cb = base + c * _GCH
            if c >= 2:
                scat[c - 2][0].wait()
                scat[c - 2][1].wait()
            loads[c] = pltpu.async_copy(x_hbm.at[pl.ds(cb, _GCH)],
                                        rows_v.at[c % 2], lsem)
            pltpu.sync_copy(slot_hbm.at[0, pl.ds(cb, _GCH)], idx0_v.at[c % 2])
            pltpu.sync_copy(slot_hbm.at[1, pl.ds(cb, _GCH)], idx1_v.at[c % 2])
            loads[c].wait()
            scat[c] = (
                pltpu.async_copy(rows_v.at[c % 2], out_hbm.at[idx0_v.at[c % 2]],
                                 sem0),
                pltpu.async_copy(rows_v.at[c % 2], out_hbm.at[idx1_v.at[c % 2]],
                                 sem1),
            )
        for c in range(max(nch - 2, 0), nch):
            scat[c][0].wait()
            scat[c][1].wait()

    @functools.partial(
        pl.kernel, mesh=mesh,
        out_type=jax.ShapeDtypeStruct((A, H), jnp.float32),
        scratch_types=[
            pltpu.VMEM((2, _GCH), jnp.int32),
            pltpu.VMEM((2, _GCH, H), jnp.float32),
            pltpu.SemaphoreType.DMA,
            pltpu.SemaphoreType.DMA,
        ],
    )
    def _sc_gather_pairs(y_hbm, slot_hbm, out_hbm, idx_v, rows_v, gsem, ssem):
        wid = _sc_wid()
        per = T // NW
        nch = per // _GCH
        base = wid * per
        stores = [None] * (K * nch)
        for k in range(K):
            for c in range(nch):
                i = k * nch + c
                cb = base + c * _GCH
                if i >= 2:
                    stores[i - 2].wait()
                pltpu.sync_copy(slot_hbm.at[k, pl.ds(cb, _GCH)],
                                idx_v.at[i % 2])
                pltpu.async_copy(y_hbm.at[idx_v.at[i % 2]], rows_v.at[i % 2],
                                 gsem).wait()
                stores[i] = pltpu.async_copy(
                    rows_v.at[i % 2], out_hbm.at[pl.ds(k * T + cb, _GCH)],
                    ssem)
        for i in range(max(K * nch - 2, 0), K * nch):
            stores[i].wait()

    return _sc_scatter_x, _sc_gather_pairs


def _group_mlp_body(eid_ref, x_ref, wg_ref, wu_ref, wd_ref, y_ref):
    x = x_ref[...]
    g = jnp.dot(x, wg_ref[0], preferred_element_type=jnp.float32)
    u = jnp.dot(x, wu_ref[0], preferred_element_type=jnp.float32)
    a = g * jax.nn.sigmoid(g) * u
    y_ref[...] = jnp.dot(a, wd_ref[0], preferred_element_type=jnp.float32)


def _shared_body(x_ref, wg_ref, wu_ref, wd_ref, out_ref, acc_ref):
    f = pl.program_id(0)
    t = pl.program_id(1)
    x = x_ref[...]
    g = jnp.dot(x, wg_ref[0], preferred_element_type=jnp.float32)
    u = jnp.dot(x, wu_ref[0], preferred_element_type=jnp.float32)
    a = g * jax.nn.sigmoid(g) * u
    y = jnp.dot(a, wd_ref[0], preferred_element_type=jnp.float32)
    base = t * 256
    nf = F // FC

    @pl.when(f == 0)
    def _():
        acc_ref[pl.ds(base, 256), :] = y

    @pl.when(f == nf - 1)
    def _():
        out_ref[...] = acc_ref[pl.ds(base, 256), :] + y


def _combine_body(x_ref, sh0_ref, sh1_ref, yk0_ref, yk1_ref, tv_ref, out_ref):
    tv = tv_ref[...]
    out_ref[...] = (x_ref[...] + sh0_ref[...] + sh1_ref[...]
                    + tv[:, 0:1] * yk0_ref[...] + tv[:, 1:2] * yk1_ref[...])


def kernel(hidden_states, Wr, br, lWg, lWu, lWd, sWg, sWu, sWd):
    flat = hidden_states.reshape(T, H)

    RB = 512
    scores, tv, ti = pl.pallas_call(
        _router_body,
        grid=(T // RB,),
        in_specs=[
            pl.BlockSpec((RB, H), lambda t: (t, 0)),
            pl.BlockSpec((H, E), lambda t: (0, 0)),
            pl.BlockSpec((1, E), lambda t: (0, 0)),
        ],
        out_specs=(
            pl.BlockSpec((RB, E), lambda t: (t, 0)),
            pl.BlockSpec((RB, K), lambda t: (t, 0)),
            pl.BlockSpec((RB, K), lambda t: (t, 0)),
        ),
        out_shape=(
            jax.ShapeDtypeStruct((T, E), jnp.float32),
            jax.ShapeDtypeStruct((T, K), jnp.float32),
            jax.ShapeDtypeStruct((T, K), jnp.int32),
        ),
    )(flat, Wr, br.reshape(1, E))

    ids_r = ti.T.reshape(64, 128)
    slot_r, eid = pl.pallas_call(
        _bucket_body,
        out_shape=(
            jax.ShapeDtypeStruct((64, 128), jnp.int32),
            jax.ShapeDtypeStruct((1, 128), jnp.int32),
        ),
    )(ids_r)

    sc_scatter_x, sc_gather_pairs = _sc_kernels()
    slot_kt = slot_r.reshape(K, T)
    x_sorted = sc_scatter_x(flat, slot_kt)

    TB = 256

    def _shared_call(si):
        return pl.pallas_call(
            _shared_body,
            grid=(F // FC, T // TB),
            in_specs=[
                pl.BlockSpec((TB, H), lambda f, t: (t, 0)),
                pl.BlockSpec((1, H, FC), lambda f, t, si=si: (si, 0, f)),
                pl.BlockSpec((1, H, FC), lambda f, t, si=si: (si, 0, f)),
                pl.BlockSpec((1, FC, H), lambda f, t, si=si: (si, f, 0)),
            ],
            out_specs=pl.BlockSpec((TB, H), lambda f, t: (t, 0)),
            out_shape=jax.ShapeDtypeStruct((T, H), jnp.float32),
            scratch_shapes=[pltpu.VMEM((T, H), jnp.float32)],
        )(flat, sWg, sWu, sWd)

    shared0 = _shared_call(0)
    x_sorted, shared0 = lax.optimization_barrier((x_sorted, shared0))

    eid_arr = eid.reshape(128)[:G]
    grid_spec = pltpu.PrefetchScalarGridSpec(
        num_scalar_prefetch=1,
        grid=(G,),
        in_specs=[
            pl.BlockSpec((BLK, H), lambda g, eid_ref: (g, 0)),
            pl.BlockSpec((1, H, F), lambda g, eid_ref: (eid_ref[g], 0, 0)),
            pl.BlockSpec((1, H, F), lambda g, eid_ref: (eid_ref[g], 0, 0)),
            pl.BlockSpec((1, F, H), lambda g, eid_ref: (eid_ref[g], 0, 0)),
        ],
        out_specs=pl.BlockSpec((BLK, H), lambda g, eid_ref: (g, 0)),
    )
    y_sorted = pl.pallas_call(
        _group_mlp_body,
        grid_spec=grid_spec,
        out_shape=jax.ShapeDtypeStruct((NPAD, H), jnp.float32),
    )(eid_arr, x_sorted, lWg, lWu, lWd)

    y_pair = sc_gather_pairs(y_sorted, slot_kt)
    shared1 = _shared_call(1)
    y_pair, shared1 = lax.optimization_barrier((y_pair, shared1))

    nt = T // TB
    out = pl.pallas_call(
        _combine_body,
        grid=(nt,),
        in_specs=[
            pl.BlockSpec((TB, H), lambda t: (t, 0)),
            pl.BlockSpec((TB, H), lambda t: (t, 0)),
            pl.BlockSpec((TB, H), lambda t: (t, 0)),
            pl.BlockSpec((TB, H), lambda t: (t, 0)),
            pl.BlockSpec((TB, H), lambda t: (t + nt, 0)),
            pl.BlockSpec((TB, K), lambda t: (t, 0)),
        ],
        out_specs=pl.BlockSpec((TB, H), lambda t: (t, 0)),
        out_shape=jax.ShapeDtypeStruct((T, H), jnp.float32),
    )(flat, shared0, shared1, y_pair, y_pair, tv)

    return out.reshape(B, S, H), scores.reshape(B, S, E)


# speedup vs baseline: 1.0429x; 1.0105x over previous
import functools

import jax
import jax.numpy as jnp
from jax import lax
from jax.experimental import pallas as pl
from jax.experimental.pallas import tpu as pltpu
from jax.experimental.pallas import tpu_sc as plsc

B, S, H, F, E, K, NS = 2, 2048, 1024, 2048, 8, 2, 2
T = B * S
A = T * K
BLK = 256
G = (A + E * (BLK - 1)) // BLK + 1
NPAD = G * BLK
FC = 1024


def _router_body(x_ref, wr_ref, br_ref, scores_ref, tv_ref, ti_ref):
    x = x_ref[...]
    scores = jnp.dot(x, wr_ref[...], preferred_element_type=jnp.float32)
    scores = scores + br_ref[...]
    scores_ref[...] = scores
    m = jnp.max(scores, axis=-1, keepdims=True)
    ex = jnp.exp(scores - m)
    probs = ex / jnp.sum(ex, axis=-1, keepdims=True)
    cols = lax.broadcasted_iota(jnp.int32, probs.shape, 1)
    v1 = jnp.max(probs, axis=-1, keepdims=True)
    i1 = jnp.min(jnp.where(probs == v1, cols, jnp.int32(E)), axis=-1,
                 keepdims=True)
    probs2 = jnp.where(cols == i1, jnp.float32(-jnp.inf), probs)
    v2 = jnp.max(probs2, axis=-1, keepdims=True)
    i2 = jnp.min(jnp.where(probs2 == v2, cols, jnp.int32(E)), axis=-1,
                 keepdims=True)
    tv_ref[...] = jnp.concatenate([v1, v2], axis=1)
    ti_ref[...] = jnp.concatenate([i1, i2], axis=1)


def _bucket_body(ids_ref, slot_ref, eid_ref):
    ids = ids_ref[...]
    r_ = lax.broadcasted_iota(jnp.int32, (128, 128), 0)
    c_ = lax.broadcasted_iota(jnp.int32, (128, 128), 1)
    ustrict = jnp.where(r_ < c_, 1.0, 0.0)
    r64 = lax.broadcasted_iota(jnp.int32, (64, 64), 0)
    c64 = lax.broadcasted_iota(jnp.int32, (64, 64), 1)
    lstrict = jnp.where(c64 < r64, 1.0, 0.0)

    counts = []
    masks = []
    for e in range(E):
        m = (ids == e).astype(jnp.float32)
        masks.append(m)
        counts.append(jnp.sum(m))
    offs = []
    cums = []
    acc = jnp.float32(0.0)
    for e in range(E):
        offs.append(acc)
        pc = jnp.floor((counts[e] + (BLK - 1)) / BLK) * BLK
        acc = acc + pc
        cums.append(acc)

    slot = jnp.zeros((64, 128), jnp.float32)
    for e in range(E):
        m = masks[e]
        within = jnp.dot(m, ustrict, preferred_element_type=jnp.float32)
        s = jnp.sum(m, axis=1, keepdims=True)
        p = jnp.dot(lstrict, s, preferred_element_type=jnp.float32)
        rank = within + p
        slot = slot + m * (offs[e] + rank)
    slot_ref[...] = slot.astype(jnp.int32)

    gbase = lax.broadcasted_iota(jnp.int32, (1, 128), 1).astype(jnp.float32)
    gbase = gbase * BLK
    eid = jnp.zeros((1, 128), jnp.float32)
    for e in range(E):
        eid = eid + jnp.where(gbase >= cums[e], 1.0, 0.0)
    eid_ref[...] = jnp.minimum(eid, E - 1).astype(jnp.int32)


NW = 32
_GCH = 32


def _sc_wid():
    return lax.axis_index("s") * 2 + lax.axis_index("c")


@functools.lru_cache(maxsize=1)
def _sc_kernels():
    mesh = plsc.VectorSubcoreMesh(core_axis_name="c", subcore_axis_name="s")

    @functools.partial(
        pl.kernel, mesh=mesh,
        out_type=jax.ShapeDtypeStruct((NPAD, H), jnp.float32),
        scratch_types=[
            pltpu.VMEM((2, _GCH), jnp.int32),
            pltpu.VMEM((2, _GCH), jnp.int32),
            pltpu.VMEM((2, _GCH, H), jnp.float32),
            pltpu.SemaphoreType.DMA,
            pltpu.SemaphoreType.DMA,
            pltpu.SemaphoreType.DMA,
        ],
    )
    def _sc_scatter_x(x_hbm, slot_hbm, out_hbm, idx0_v, idx1_v, rows_v,
                      lsem, sem0, sem1):
        wid = _sc_wid()
        per = T // NW
        base = wid * per
        nch = per // _GCH
        loads = [None] * nch
        scat = [None] * nch
        for c in range(nch):
            cb = base + c * _GCH
            if c >= 2:
                scat[c - 2][0].wait()
                scat[c - 2][1].wait()
            loads[c] = pltpu.async_copy(x_hbm.at[pl.ds(cb, _GCH)],
                                        rows_v.at[c % 2], lsem)
            pltpu.sync_copy(slot_hbm.at[0, pl.ds(cb, _GCH)], idx0_v.at[c % 2])
            pltpu.sync_copy(slot_hbm.at[1, pl.ds(cb, _GCH)], idx1_v.at[c % 2])
            loads[c].wait()
            scat[c] = (
                pltpu.async_copy(rows_v.at[c % 2], out_hbm.at[idx0_v.at[c % 2]],
                                 sem0),
                pltpu.async_copy(rows_v.at[c % 2], out_hbm.at[idx1_v.at[c % 2]],
                                 sem1),
            )
        for c in range(max(nch - 2, 0), nch):
            scat[c][0].wait()
            scat[c][1].wait()

    @functools.partial(
        pl.kernel, mesh=mesh,
        out_type=jax.ShapeDtypeStruct((A, H), jnp.float32),
        scratch_types=[
            pltpu.VMEM((2, _GCH), jnp.int32),
            pltpu.VMEM((2, _GCH, H), jnp.float32),
            pltpu.SemaphoreType.DMA,
            pltpu.SemaphoreType.DMA,
        ],
    )
    def _sc_gather_pairs(y_hbm, slot_hbm, out_hbm, idx_v, rows_v, gsem, ssem):
        wid = _sc_wid()
        per = T // NW
        nch = per // _GCH
        base = wid * per
        stores = [None] * (K * nch)
        for k in range(K):
            for c in range(nch):
                i = k * nch + c
                cb = base + c * _GCH
                if i >= 2:
                    stores[i - 2].wait()
                pltpu.sync_copy(slot_hbm.at[k, pl.ds(cb, _GCH)],
                                idx_v.at[i % 2])
                pltpu.async_copy(y_hbm.at[idx_v.at[i % 2]], rows_v.at[i % 2],
                                 gsem).wait()
                stores[i] = pltpu.async_copy(
                    rows_v.at[i % 2], out_hbm.at[pl.ds(k * T + cb, _GCH)],
                    ssem)
        for i in range(max(K * nch - 2, 0), K * nch):
            stores[i].wait()

    return _sc_scatter_x, _sc_gather_pairs


def _group_mlp_body(eid_ref, x_ref, wg_ref, wu_ref, wd_ref, y_ref):
    x = x_ref[...]
    g = jnp.dot(x, wg_ref[0], preferred_element_type=jnp.float32)
    u = jnp.dot(x, wu_ref[0], preferred_element_type=jnp.float32)
    a = g * jax.nn.sigmoid(g) * u
    y_ref[...] = jnp.dot(a, wd_ref[0], preferred_element_type=jnp.float32)


def _make_shared_body(add_x):
    def _shared_body(x_ref, wg_ref, wu_ref, wd_ref, out_ref, acc_ref):
        f = pl.program_id(0)
        t = pl.program_id(1)
        x = x_ref[...]
        g = jnp.dot(x, wg_ref[0], preferred_element_type=jnp.float32)
        u = jnp.dot(x, wu_ref[0], preferred_element_type=jnp.float32)
        a = g * jax.nn.sigmoid(g) * u
        y = jnp.dot(a, wd_ref[0], preferred_element_type=jnp.float32)
        base = t * 256
        nf = F // FC

        @pl.when(f == 0)
        def _():
            acc_ref[pl.ds(base, 256), :] = y

        @pl.when(f == nf - 1)
        def _():
            res = acc_ref[pl.ds(base, 256), :] + y
            out_ref[...] = res + x if add_x else res

    return _shared_body


def _combine_body(sh0_ref, sh1_ref, yk0_ref, yk1_ref, tv_ref, out_ref):
    tv = tv_ref[...]
    out_ref[...] = (sh0_ref[...] + sh1_ref[...]
                    + tv[:, 0:1] * yk0_ref[...] + tv[:, 1:2] * yk1_ref[...])


def kernel(hidden_states, Wr, br, lWg, lWu, lWd, sWg, sWu, sWd):
    flat = hidden_states.reshape(T, H)

    RB = 512
    scores, tv, ti = pl.pallas_call(
        _router_body,
        grid=(T // RB,),
        in_specs=[
            pl.BlockSpec((RB, H), lambda t: (t, 0)),
            pl.BlockSpec((H, E), lambda t: (0, 0)),
            pl.BlockSpec((1, E), lambda t: (0, 0)),
        ],
        out_specs=(
            pl.BlockSpec((RB, E), lambda t: (t, 0)),
            pl.BlockSpec((RB, K), lambda t: (t, 0)),
            pl.BlockSpec((RB, K), lambda t: (t, 0)),
        ),
        out_shape=(
            jax.ShapeDtypeStruct((T, E), jnp.float32),
            jax.ShapeDtypeStruct((T, K), jnp.float32),
            jax.ShapeDtypeStruct((T, K), jnp.int32),
        ),
    )(flat, Wr, br.reshape(1, E))

    ids_r = ti.T.reshape(64, 128)
    slot_r, eid = pl.pallas_call(
        _bucket_body,
        out_shape=(
            jax.ShapeDtypeStruct((64, 128), jnp.int32),
            jax.ShapeDtypeStruct((1, 128), jnp.int32),
        ),
    )(ids_r)

    sc_scatter_x, sc_gather_pairs = _sc_kernels()
    slot_kt = slot_r.reshape(K, T)
    x_sorted = sc_scatter_x(flat, slot_kt)

    TB = 256

    def _shared_call(si):
        return pl.pallas_call(
            _make_shared_body(si == 0),
            grid=(F // FC, T // TB),
            in_specs=[
                pl.BlockSpec((TB, H), lambda f, t: (t, 0)),
                pl.BlockSpec((1, H, FC), lambda f, t, si=si: (si, 0, f)),
                pl.BlockSpec((1, H, FC), lambda f, t, si=si: (si, 0, f)),
                pl.BlockSpec((1, FC, H), lambda f, t, si=si: (si, f, 0)),
            ],
            out_specs=pl.BlockSpec((TB, H), lambda f, t: (t, 0)),
            out_shape=jax.ShapeDtypeStruct((T, H), jnp.float32),
            scratch_shapes=[pltpu.VMEM((T, H), jnp.float32)],
        )(flat, sWg, sWu, sWd)

    shared0 = _shared_call(0)
    x_sorted, shared0 = lax.optimization_barrier((x_sorted, shared0))

    eid_arr = eid.reshape(128)[:G]
    grid_spec = pltpu.PrefetchScalarGridSpec(
        num_scalar_prefetch=1,
        grid=(G,),
        in_specs=[
            pl.BlockSpec((BLK, H), lambda g, eid_ref: (g, 0)),
            pl.BlockSpec((1, H, F), lambda g, eid_ref: (eid_ref[g], 0, 0)),
            pl.BlockSpec((1, H, F), lambda g, eid_ref: (eid_ref[g], 0, 0)),
            pl.BlockSpec((1, F, H), lambda g, eid_ref: (eid_ref[g], 0, 0)),
        ],
        out_specs=pl.BlockSpec((BLK, H), lambda g, eid_ref: (g, 0)),
    )
    y_sorted = pl.pallas_call(
        _group_mlp_body,
        grid_spec=grid_spec,
        out_shape=jax.ShapeDtypeStruct((NPAD, H), jnp.float32),
    )(eid_arr, x_sorted, lWg, lWu, lWd)

    y_pair = sc_gather_pairs(y_sorted, slot_kt)
    shared1 = _shared_call(1)
    y_pair, shared1 = lax.optimization_barrier((y_pair, shared1))

    nt = T // TB
    out = pl.pallas_call(
        _combine_body,
        grid=(nt,),
        in_specs=[
            pl.BlockSpec((TB, H), lambda t: (t, 0)),
            pl.BlockSpec((TB, H), lambda t: (t, 0)),
            pl.BlockSpec((TB, H), lambda t: (t, 0)),
            pl.BlockSpec((TB, H), lambda t: (t + nt, 0)),
            pl.BlockSpec((TB, K), lambda t: (t, 0)),
        ],
        out_specs=pl.BlockSpec((TB, H), lambda t: (t, 0)),
        out_shape=jax.ShapeDtypeStruct((T, H), jnp.float32),
    )(shared0, shared1, y_pair, y_pair, tv)

    return out.reshape(B, S, H), scores.reshape(B, S, E)
